# speedup vs baseline: 1.0230x; 1.0230x over previous
import jax
import jax.numpy as jnp
from jax.experimental import pallas as pl


def _layer_norm(h, eps=1e-5):
    mu = jnp.mean(h, axis=-1, keepdims=True)
    var = jnp.mean((h - mu) ** 2, axis=-1, keepdims=True)
    return (h - mu) / jnp.sqrt(var + eps)


def _pt_conv(x, pos, src, dst, n, p):
    a_src = x @ p['W_src']
    a_dst = x @ p['W_dst']
    xv = x @ p['W_lin']
    delta = (pos[dst] - pos[src]) @ p['W_pos'] + p['b_pos']
    alpha = a_dst[dst] - a_src[src] + delta
    amax = jax.ops.segment_max(alpha, dst, num_segments=n)
    ex = jnp.exp(alpha - amax[dst])
    denom = jax.ops.segment_sum(ex, dst, num_segments=n)
    attn = ex / (denom[dst] + 1e-16)
    msg = attn * (xv[src] + delta)
    return jax.ops.segment_sum(msg, dst, num_segments=n)


def _mlp_block(z_ref, w1, b1, w2, b2, w3, b3, w4, b4, s_ref, o_ref):
    h = jnp.maximum(z_ref[...] @ w1[...] + b1[...], 0.0)
    h = jnp.maximum(h @ w2[...] + b2[...], 0.0)
    h = jnp.maximum(h @ w3[...] + b3[...], 0.0)
    o_ref[...] = (h @ w4[...] + b4[...]) * s_ref[...]


def _mlp_head(z, m, scale):
    n = z.shape[0]
    blk = 2000
    grid = (n + blk - 1) // blk
    full = lambda *s: pl.BlockSpec(s, lambda i: tuple(0 for _ in s))
    out = pl.pallas_call(
        _mlp_block,
        grid=(grid,),
        in_specs=[
            pl.BlockSpec((blk, 320), lambda i: (i, 0)),
            full(320, 128), full(1, 128),
            full(128, 128), full(1, 128),
            full(128, 64), full(1, 64),
            full(64, 2), full(1, 2), full(1, 2),
        ],
        out_specs=pl.BlockSpec((blk, 2), lambda i: (i, 0)),
        out_shape=jax.ShapeDtypeStruct((grid * blk, 2), jnp.float32),
    )(z, m['W1'], m['b1'][None], m['W2'], m['b2'][None],
      m['W3'], m['b3'][None], m['W4'], m['b4'][None], scale[None])
    return out[:n]


def kernel(x, pos, edge_index, params, scale):
    n = x.shape[0]
    loops = jnp.arange(n, dtype=edge_index.dtype)
    src = jnp.concatenate([edge_index[0], loops])
    dst = jnp.concatenate([edge_index[1], loops])
    embeddings = []
    h_in = x
    for i in range(5):
        if i == 0:
            inp = jnp.concatenate([h_in, pos / 345.0], axis=1)
            h = _pt_conv(inp, pos, src, dst, n, params['conv0'])
        else:
            h = _pt_conv(h_in, pos, src, dst, n, params['convs'][i - 1])
        g, b = params['norms'][i]
        h = _layer_norm(h) * g + b
        embeddings.append(h)
        h_in = jax.nn.relu(h + h_in) if i > 0 else jax.nn.relu(h)
    z = jnp.concatenate(embeddings, axis=-1)
    z = _layer_norm(z)
    return _mlp_head(z, params['mlp'], scale)


# speedup vs baseline: 3.6946x; 3.6115x over previous
import functools

import jax
import jax.numpy as jnp
from jax import lax
from jax.experimental import pallas as pl
from jax.experimental.pallas import tpu as pltpu
from jax.experimental.pallas import tpu_sc as plsc

NW = 32
K = 64
KP = K + 8
NPT = 1563


def _sread(ref, i):
    return ref[pl.ds(i, 16)][0]


def _edge_kernel_body(ru_hbm, src_hbm, dst_hbm, delta_hbm, meta_hbm,
                      out_hbm, metav, idxv, dstv, dlrows, rurows, outbuf,
                      sem):
    wid = lax.axis_index("s") * 2 + lax.axis_index("c")
    pltpu.sync_copy(meta_hbm, metav)
    e0 = _sread(metav, wid)
    e1 = _sread(metav, wid + 1)
    base = wid * NPT

    ninf = jnp.float32(-jnp.inf)
    m_init = tuple(jnp.full((16,), ninf, jnp.float32) for _ in range(4))
    z_init = tuple(jnp.zeros((16,), jnp.float32) for _ in range(4))

    def chunk_body(i, carry):
        m, s, acc, d_prev = carry
        eb = e0 + i * K
        eb_al = (eb // 8) * 8
        j0 = eb - eb_al
        cnt = jnp.minimum(jnp.int32(K), e1 - eb)
        pltpu.sync_copy(src_hbm.at[pl.ds(eb_al, KP)], idxv)
        pltpu.sync_copy(dst_hbm.at[pl.ds(eb_al, KP + 16)], dstv)
        pltpu.sync_copy(delta_hbm.at[pl.ds(eb_al, KP), :], dlrows)
        pltpu.async_copy(ru_hbm.at[idxv], rurows, sem).wait()

        def edge_body(j, ec):
            m, s, acc, d_prev = ec
            d_j = _sread(dstv, j)
            bnew = d_j != d_prev
            off = (d_j - base) * 64
            nm, ns, na = [], [], []
            for c in range(4):
                av = rurows[j, pl.ds(16 * c, 16)]
                xvv = rurows[j, pl.ds(64 + 16 * c, 16)]
                delta = dlrows[j, pl.ds(16 * c, 16)]
                alpha = delta - av
                m_pre = jnp.where(bnew, ninf, m[c])
                s_pre = jnp.where(bnew, 0.0, s[c])
                a_pre = jnp.where(bnew, 0.0, acc[c])
                m2 = jnp.maximum(m_pre, alpha)
                rescale = jnp.exp(m_pre - m2)
                e = jnp.exp(alpha - m2)
                v = xvv + delta
                s2 = s_pre * rescale + e
                a2 = a_pre * rescale + e * v
                outbuf[pl.ds(off + 16 * c, 16)] = a2 / (s2 + 1e-16)
                nm.append(m2)
                ns.append(s2)
                na.append(a2)
            return tuple(nm), tuple(ns), tuple(na), d_j

        return lax.fori_loop(j0, j0 + cnt, edge_body, (m, s, acc, d_prev))

    @pl.when(e1 > e0)
    def _():
        nch = (e1 - e0 + K - 1) // K
        lax.fori_loop(0, nch, chunk_body,
                      (m_init, z_init, z_init, jnp.int32(-1)))
    pltpu.sync_copy(outbuf, out_hbm.at[pl.ds(base * 64, NPT * 64)])


def _edge_phase(ru, src_p, dst_p, delta_p, meta, n):
    mesh = plsc.VectorSubcoreMesh(core_axis_name="c", subcore_axis_name="s")
    kfn = functools.partial(
        pl.kernel,
        mesh=mesh,
        out_type=jax.ShapeDtypeStruct((NW * NPT * 64,), jnp.float32),
        scratch_types=[
            pltpu.VMEM((48,), jnp.int32),
            pltpu.VMEM((KP,), jnp.int32),
            pltpu.VMEM((KP + 16,), jnp.int32),
            pltpu.VMEM((KP, 64), jnp.float32),
            pltpu.VMEM((KP, 128), jnp.float32),
            pltpu.VMEM((NPT * 64,), jnp.float32),
            pltpu.SemaphoreType.DMA,
        ],
    )(_edge_kernel_body)
    out = kfn(ru, src_p, dst_p, delta_p, meta)
    return out.reshape(NW * NPT, 64)[:n]


def _prep_edges(x, pos, edge_index, n):
    loops = jnp.arange(n, dtype=jnp.int32)
    src = jnp.concatenate([edge_index[0], loops])
    dst = jnp.concatenate([edge_index[1], loops])
    dst_s, src_s = lax.sort_key_val(dst, src)
    bases = jnp.minimum(jnp.arange(NW + 1, dtype=jnp.int32) * NPT, n)
    bounds = jnp.searchsorted(dst_s, bases, side='left').astype(jnp.int32)
    meta = jnp.concatenate([bounds, jnp.zeros((48 - NW - 1,), jnp.int32)])
    pdiff = pos[dst_s] - pos[src_s]
    pad = KP + 24
    src_p = jnp.concatenate([src_s, jnp.zeros((pad,), jnp.int32)])
    dst_p = jnp.concatenate([dst_s, jnp.full((pad,), n, jnp.int32)])
    return src_p, dst_p, pdiff, meta


def _layer_norm(h, eps=1e-5):
    mu = jnp.mean(h, axis=-1, keepdims=True)
    var = jnp.mean((h - mu) ** 2, axis=-1, keepdims=True)
    return (h - mu) / jnp.sqrt(var + eps)


def _pt_conv_sc(h, src_p, dst_p, pdiff, meta, n, p):
    a_src = h @ p['W_src']
    xv = h @ p['W_lin']
    ru = jnp.concatenate([a_src, xv], axis=1)
    ru = jnp.concatenate([ru, jnp.zeros((8, 128), jnp.float32)])
    delta = pdiff @ p['W_pos'] + p['b_pos']
    delta_p = jnp.concatenate(
        [delta, jnp.zeros((KP + 24, 64), jnp.float32)])
    return _edge_phase(ru, src_p, dst_p, delta_p, meta, n)


def _mlp_block(z_ref, w1, b1, w2, b2, w3, b3, w4, b4, s_ref, o_ref):
    h = jnp.maximum(z_ref[...] @ w1[...] + b1[...], 0.0)
    h = jnp.maximum(h @ w2[...] + b2[...], 0.0)
    h = jnp.maximum(h @ w3[...] + b3[...], 0.0)
    o_ref[...] = (h @ w4[...] + b4[...]) * s_ref[...]


def _mlp_head(z, m, scale):
    n = z.shape[0]
    blk = 2000
    grid = (n + blk - 1) // blk
    full = lambda *s: pl.BlockSpec(s, lambda i: tuple(0 for _ in s))
    out = pl.pallas_call(
        _mlp_block,
        grid=(grid,),
        in_specs=[
            pl.BlockSpec((blk, 320), lambda i: (i, 0)),
            full(320, 128), full(1, 128),
            full(128, 128), full(1, 128),
            full(128, 64), full(1, 64),
            full(64, 2), full(1, 2), full(1, 2),
        ],
        out_specs=pl.BlockSpec((blk, 2), lambda i: (i, 0)),
        out_shape=jax.ShapeDtypeStruct((grid * blk, 2), jnp.float32),
    )(z, m['W1'], m['b1'][None], m['W2'], m['b2'][None],
      m['W3'], m['b3'][None], m['W4'], m['b4'][None], scale[None])
    return out[:n]


def kernel(x, pos, edge_index, params, scale):
    n = x.shape[0]
    src_p, dst_p, pdiff, meta = _prep_edges(x, pos, edge_index, n)
    embeddings = []
    h_in = x
    for i in range(5):
        p = params['conv0'] if i == 0 else params['convs'][i - 1]
        if i == 0:
            inp = jnp.concatenate([h_in, pos / 345.0], axis=1)
            h = _pt_conv_sc(inp, src_p, dst_p, pdiff, meta, n, p)
        else:
            h = _pt_conv_sc(h_in, src_p, dst_p, pdiff, meta, n, p)
        g, b = params['norms'][i]
        h = _layer_norm(h) * g + b
        embeddings.append(h)
        h_in = jax.nn.relu(h + h_in) if i > 0 else jax.nn.relu(h)
    z = jnp.concatenate(embeddings, axis=-1)
    z = _layer_norm(z)
    return _mlp_head(z, params['mlp'], scale)
